# Initial kernel scaffold; baseline (speedup 1.0000x reference)
import jax
import jax.numpy as jnp
from jax.experimental import pallas as pl


def kernel(idx, weight):
    raise NotImplementedError("write your pallas kernel here")


# speedup vs baseline: 5.0061x; 5.0061x over previous
import functools

import jax
import jax.numpy as jnp
from jax import lax
from jax.experimental import pallas as pl
from jax.experimental.pallas import tpu as pltpu
from jax.experimental.pallas import tpu_sc as plsc

_D = 32
_B = 16384 * 200
_NC = 2
_NS = 16
_NW = _NC * _NS
_BPW = _B // _NW
_CH = 3_200
_NCHUNK = _BPW // _CH

_mesh = plsc.VectorSubcoreMesh(core_axis_name="c", subcore_axis_name="s")


@functools.partial(
    pl.kernel,
    out_type=jax.ShapeDtypeStruct((_B, _D), jnp.float32),
    mesh=_mesh,
    scratch_types=[
        pltpu.VMEM((_CH,), jnp.int32),
        pltpu.VMEM((_CH, _D), jnp.float32),
        pltpu.SemaphoreType.DMA,
    ],
    compiler_params=pltpu.CompilerParams(use_tc_tiling_on_sc=False),
)
def _gather_kernel(idx_hbm, tab_hbm, out_hbm, idx_v, rows_v, sem):
  wid = lax.axis_index("s") * _NC + lax.axis_index("c")
  base = wid * _BPW

  def body(g, carry):
    off = base + g * _CH
    pltpu.sync_copy(idx_hbm.at[pl.ds(off, _CH)], idx_v)
    pltpu.async_copy(tab_hbm.at[idx_v], rows_v, sem).wait()
    pltpu.sync_copy(rows_v, out_hbm.at[pl.ds(off, _CH)])
    return carry

  lax.fori_loop(0, _NCHUNK, body, 0)


def kernel(idx, weight):
  flat = idx.reshape(-1).astype(jnp.int32)
  out = _gather_kernel(flat, weight)
  return out.reshape(idx.shape + (_D,))


# speedup vs baseline: 5.0356x; 1.0059x over previous
import functools

import jax
import jax.numpy as jnp
from jax import lax
from jax.experimental import pallas as pl
from jax.experimental.pallas import tpu as pltpu
from jax.experimental.pallas import tpu_sc as plsc

_D = 32
_B = 16384 * 200
_NC = 2
_NS = 16
_NW = _NC * _NS
_BPW = _B // _NW
_CH = 1_600
_NBUF = 2
_NCHUNK = _BPW // _CH

_mesh = plsc.VectorSubcoreMesh(core_axis_name="c", subcore_axis_name="s")


@functools.partial(
    pl.kernel,
    out_type=jax.ShapeDtypeStruct((_B, _D), jnp.float32),
    mesh=_mesh,
    scratch_types=[
        pltpu.VMEM((_NBUF, _CH), jnp.int32),
        pltpu.VMEM((_NBUF, _CH, _D), jnp.float32),
        pltpu.SemaphoreType.DMA((_NBUF,)),
        pltpu.SemaphoreType.DMA((_NBUF,)),
        pltpu.SemaphoreType.DMA((_NBUF,)),
    ],
    compiler_params=pltpu.CompilerParams(use_tc_tiling_on_sc=False),
)
def _gather_kernel(idx_hbm, tab_hbm, out_hbm, idx_v, rows_v, sem_i, sem_g,
                   sem_o):
  wid = lax.axis_index("s") * _NC + lax.axis_index("c")
  base = wid * _BPW

  def start_idx(g, b):
    pltpu.async_copy(
        idx_hbm.at[pl.ds(base + g * _CH, _CH)], idx_v.at[b], sem_i.at[b])

  def wait_idx(b):
    pltpu.make_async_copy(
        idx_hbm.at[pl.ds(0, _CH)], idx_v.at[b], sem_i.at[b]).wait()

  def start_gather(b):
    pltpu.async_copy(tab_hbm.at[idx_v.at[b]], rows_v.at[b], sem_g.at[b])

  def wait_gather(b):
    pltpu.make_async_copy(
        tab_hbm.at[idx_v.at[b]], rows_v.at[b], sem_g.at[b]).wait()

  def start_out(g, b):
    pltpu.async_copy(
        rows_v.at[b], out_hbm.at[pl.ds(base + g * _CH, _CH)], sem_o.at[b])

  def wait_out(b):
    pltpu.make_async_copy(
        rows_v.at[b], out_hbm.at[pl.ds(0, _CH)], sem_o.at[b]).wait()

  for b in range(_NBUF):
    start_idx(b, b)

  @pl.loop(0, _NCHUNK, step=_NBUF)
  def _outer(g0):
    for b in range(_NBUF):
      g = g0 + b
      wait_idx(b)

      @pl.when(g >= _NBUF)
      def _():
        wait_out(b)

      start_gather(b)
      wait_gather(b)
      start_out(g, b)

      @pl.when(g + _NBUF < _NCHUNK)
      def _():
        start_idx(g + _NBUF, b)

  for b in range(_NBUF):
    wait_out(b)


def kernel(idx, weight):
  flat = idx.reshape(-1).astype(jnp.int32)
  out = _gather_kernel(flat, weight)
  return out.reshape(idx.shape + (_D,))


# speedup vs baseline: 5.0543x; 1.0037x over previous
import functools

import jax
import jax.numpy as jnp
from jax import lax
from jax.experimental import pallas as pl
from jax.experimental.pallas import tpu as pltpu
from jax.experimental.pallas import tpu_sc as plsc

_D = 32
_B = 16384 * 200
_NC = 2
_NS = 16
_NW = _NC * _NS
_BPW = _B // _NW
_CH = 800
_NBUF = 4
_NCHUNK = _BPW // _CH

_mesh = plsc.VectorSubcoreMesh(core_axis_name="c", subcore_axis_name="s")


@functools.partial(
    pl.kernel,
    out_type=jax.ShapeDtypeStruct((_B, _D), jnp.float32),
    mesh=_mesh,
    scratch_types=[
        pltpu.VMEM((_NBUF, _CH), jnp.int32),
        pltpu.VMEM((_NBUF, _CH, _D), jnp.float32),
        pltpu.SemaphoreType.DMA((_NBUF,)),
        pltpu.SemaphoreType.DMA((_NBUF,)),
        pltpu.SemaphoreType.DMA((_NBUF,)),
    ],
    compiler_params=pltpu.CompilerParams(use_tc_tiling_on_sc=False),
)
def _gather_kernel(idx_hbm, tab_hbm, out_hbm, idx_v, rows_v, sem_i, sem_g,
                   sem_o):
  wid = lax.axis_index("s") * _NC + lax.axis_index("c")
  base = wid * _BPW

  def start_idx(g, b):
    pltpu.async_copy(
        idx_hbm.at[pl.ds(base + g * _CH, _CH)], idx_v.at[b], sem_i.at[b])

  def wait_idx(b):
    pltpu.make_async_copy(
        idx_hbm.at[pl.ds(0, _CH)], idx_v.at[b], sem_i.at[b]).wait()

  def start_gather(b):
    pltpu.async_copy(tab_hbm.at[idx_v.at[b]], rows_v.at[b], sem_g.at[b])

  def wait_gather(b):
    pltpu.make_async_copy(
        tab_hbm.at[idx_v.at[b]], rows_v.at[b], sem_g.at[b]).wait()

  def start_out(g, b):
    pltpu.async_copy(
        rows_v.at[b], out_hbm.at[pl.ds(base + g * _CH, _CH)], sem_o.at[b])

  def wait_out(b):
    pltpu.make_async_copy(
        rows_v.at[b], out_hbm.at[pl.ds(0, _CH)], sem_o.at[b]).wait()

  for b in range(_NBUF):
    start_idx(b, b)
  wait_idx(0)
  start_gather(0)

  @pl.loop(0, _NCHUNK, step=_NBUF)
  def _outer(g0):
    for b in range(_NBUF):
      g = g0 + b
      bn = (b + 1) % _NBUF

      @pl.when(g + 1 < _NCHUNK)
      def _():
        wait_idx(bn)

        @pl.when(g + 1 >= _NBUF)
        def _():
          wait_out(bn)

        start_gather(bn)

      wait_gather(b)
      start_out(g, b)

      @pl.when(g + _NBUF < _NCHUNK)
      def _():
        start_idx(g + _NBUF, b)

  for b in range(_NBUF):
    wait_out(b)


def kernel(idx, weight):
  flat = idx.reshape(-1).astype(jnp.int32)
  out = _gather_kernel(flat, weight)
  return out.reshape(idx.shape + (_D,))


# speedup vs baseline: 6.6955x; 1.3247x over previous
import functools

import jax
import jax.numpy as jnp
from jax import lax
from jax.experimental import pallas as pl
from jax.experimental.pallas import tpu as pltpu
from jax.experimental.pallas import tpu_sc as plsc

_D = 32
_B = 16384 * 200
_NC = 2
_NS = 16
_NW = _NC * _NS
_BPW = _B // _NW
_CH = 800
_NBUF = 4
_NCHUNK = _BPW // _CH

_mesh = plsc.VectorSubcoreMesh(core_axis_name="c", subcore_axis_name="s")


@functools.partial(
    pl.kernel,
    out_type=jax.ShapeDtypeStruct((_B, _D), jnp.float32),
    mesh=_mesh,
    scratch_types=[
        pltpu.VMEM((_NBUF, _CH), jnp.int32),
        pltpu.VMEM((_NBUF, _CH, _D), jnp.float32),
        pltpu.SemaphoreType.DMA((_NBUF,)),
        pltpu.SemaphoreType.DMA((_NBUF,)),
        pltpu.SemaphoreType.DMA((_NBUF,)),
    ],
    compiler_params=pltpu.CompilerParams(use_tc_tiling_on_sc=False),
)
def _gather_kernel(idx_hbm, tab_hbm, out_hbm, idx_v, rows_v, sem_i, sem_g,
                   sem_o):
  wid = lax.axis_index("s") * _NC + lax.axis_index("c")
  base = wid * _BPW

  def start_idx(g, b):
    pltpu.async_copy(
        idx_hbm.at[pl.ds(base + g * _CH, _CH)], idx_v.at[b], sem_i.at[b])

  def wait_idx(b):
    pltpu.make_async_copy(
        idx_hbm.at[pl.ds(0, _CH)], idx_v.at[b], sem_i.at[b]).wait()

  def start_gather(b):
    pltpu.async_copy(tab_hbm.at[idx_v.at[b]], rows_v.at[b], sem_g.at[b])

  def wait_gather(b):
    pltpu.make_async_copy(
        tab_hbm.at[idx_v.at[b]], rows_v.at[b], sem_g.at[b]).wait()

  def start_out(g, b):
    pltpu.async_copy(
        rows_v.at[b], out_hbm.at[pl.ds(base + g * _CH, _CH)], sem_o.at[b])

  def wait_out(b):
    pltpu.make_async_copy(
        rows_v.at[b], out_hbm.at[pl.ds(0, _CH)], sem_o.at[b]).wait()

  for b in range(_NBUF):
    start_idx(b, b)
  wait_idx(0)
  start_gather(0)

  @pl.loop(0, _NCHUNK, step=_NBUF)
  def _outer(g0):
    for b in range(_NBUF):
      g = g0 + b
      bn = (b + 1) % _NBUF

      @pl.when(g + 1 < _NCHUNK)
      def _():
        wait_idx(bn)

        @pl.when(g + 1 >= _NBUF)
        def _():
          wait_out(bn)

        start_gather(bn)

      wait_gather(b)
      start_out(g, b)

      @pl.when(g + _NBUF < _NCHUNK)
      def _():
        start_idx(g + _NBUF, b)

  for b in range(_NBUF):
    wait_out(b)


_BATCH = 16384
_HIST = 200
_HG = 4
_NHG = _HIST // _HG


def _transpose_body(in_ref, out_ref):
  for k in range(_HG):
    out_ref[k, :, :] = jnp.swapaxes(
        in_ref[:, 0, 0, k * _D:(k + 1) * _D], 0, 1)


def _transpose_kernel(rm4):
  return pl.pallas_call(
      _transpose_body,
      grid=(_NHG,),
      in_specs=[pl.BlockSpec((_BATCH, 1, 1, 128), lambda j: (0, j, 0, 0))],
      out_specs=pl.BlockSpec((_HG, _D, _BATCH), lambda j: (j, 0, 0)),
      out_shape=jax.ShapeDtypeStruct((_HIST, _D, _BATCH), jnp.float32),
  )(rm4)


def kernel(idx, weight):
  flat = idx.reshape(-1).astype(jnp.int32)
  rm = _gather_kernel(flat, weight)
  rm4 = rm.reshape(_BATCH, _NHG, 1, 128)
  out_t = _transpose_kernel(rm4)
  return out_t.transpose(2, 0, 1)


# speedup vs baseline: 11.0120x; 1.6447x over previous
import functools

import jax
import jax.numpy as jnp
from jax import lax
from jax.experimental import pallas as pl
from jax.experimental.pallas import tpu as pltpu
from jax.experimental.pallas import tpu_sc as plsc

_D = 32
_BATCH = 16384
_HIST = 200
_B = _BATCH * _HIST
_NPART = 4
_PB = _BATCH // _NPART
_QB = _PB * _HIST
_NC = 2
_NS = 16
_NW = _NC * _NS
_BPW = _QB // _NW
_CH = 800
_NBUF = 4
_NCHUNK = _BPW // _CH

_mesh = plsc.VectorSubcoreMesh(core_axis_name="c", subcore_axis_name="s")


def _make_gather(part):
  @functools.partial(
      pl.kernel,
      out_type=jax.ShapeDtypeStruct((_QB, _D), jnp.float32),
      mesh=_mesh,
      scratch_types=[
          pltpu.VMEM((_NBUF, _CH), jnp.int32),
          pltpu.VMEM((_NBUF, _CH, _D), jnp.float32),
          pltpu.SemaphoreType.DMA((_NBUF,)),
          pltpu.SemaphoreType.DMA((_NBUF,)),
          pltpu.SemaphoreType.DMA((_NBUF,)),
      ],
      compiler_params=pltpu.CompilerParams(use_tc_tiling_on_sc=False),
  )
  def _gather_kernel(idx_hbm, tab_hbm, out_hbm, idx_v, rows_v, sem_i, sem_g,
                     sem_o):
    wid = lax.axis_index("s") * _NC + lax.axis_index("c")
    obase = wid * _BPW
    ibase = part * _QB + wid * _BPW

    def start_idx(g, b):
      pltpu.async_copy(
          idx_hbm.at[pl.ds(ibase + g * _CH, _CH)], idx_v.at[b], sem_i.at[b])

    def wait_idx(b):
      pltpu.make_async_copy(
          idx_hbm.at[pl.ds(0, _CH)], idx_v.at[b], sem_i.at[b]).wait()

    def start_gather(b):
      pltpu.async_copy(tab_hbm.at[idx_v.at[b]], rows_v.at[b], sem_g.at[b])

    def wait_gather(b):
      pltpu.make_async_copy(
          tab_hbm.at[idx_v.at[b]], rows_v.at[b], sem_g.at[b]).wait()

    def start_out(g, b):
      pltpu.async_copy(
          rows_v.at[b], out_hbm.at[pl.ds(obase + g * _CH, _CH)], sem_o.at[b])

    def wait_out(b):
      pltpu.make_async_copy(
          rows_v.at[b], out_hbm.at[pl.ds(0, _CH)], sem_o.at[b]).wait()

    for b in range(_NBUF):
      start_idx(b, b)
    wait_idx(0)
    start_gather(0)

    @pl.loop(0, _NCHUNK, step=_NBUF)
    def _outer(g0):
      for b in range(_NBUF):
        g = g0 + b
        bn = (b + 1) % _NBUF

        @pl.when(g + 1 < _NCHUNK)
        def _():
          wait_idx(bn)

          @pl.when(g + 1 >= _NBUF)
          def _():
            wait_out(bn)

          start_gather(bn)

        wait_gather(b)
        start_out(g, b)

        @pl.when(g + _NBUF < _NCHUNK)
        def _():
          start_idx(g + _NBUF, b)

    for b in range(_NBUF):
      wait_out(b)

  return _gather_kernel


_HG = 4
_NHG = _HIST // _HG


def _transpose_body(in_ref, out_ref):
  t = jnp.swapaxes(in_ref[:, 0, 0, :], 0, 1)
  out_ref[...] = t.reshape(_HG, _D, _PB)


def _transpose_body_acc(in_ref, prev_ref, out_ref):
  del prev_ref
  _transpose_body(in_ref, out_ref)


def _transpose_part(part, rm4, prev):
  in_specs = [pl.BlockSpec((_PB, 1, 1, 128), lambda j: (0, j, 0, 0))]
  args = (rm4,)
  body = _transpose_body
  aliases = {}
  if prev is not None:
    in_specs.append(pl.BlockSpec(memory_space=pl.ANY))
    args = (rm4, prev)
    body = _transpose_body_acc
    aliases = {1: 0}
  return pl.pallas_call(
      body,
      grid=(_NHG,),
      in_specs=in_specs,
      out_specs=pl.BlockSpec((_HG, _D, _PB), lambda j: (j, 0, part)),
      out_shape=jax.ShapeDtypeStruct((_HIST, _D, _BATCH), jnp.float32),
      input_output_aliases=aliases,
  )(*args)


def kernel(idx, weight):
  flat = idx.reshape(-1).astype(jnp.int32)
  out = None
  for p in range(_NPART):
    rm = _make_gather(p)(flat, weight)
    rm4 = rm.reshape(_PB, _NHG, 1, 128)
    out = _transpose_part(p, rm4, out)
  return out.transpose(2, 0, 1)
